# baseline (speedup 1.0000x reference)
import functools

import jax
import jax.numpy as jnp
from jax import lax
from jax.experimental import pallas as pl
from jax.experimental.pallas import tpu as pltpu
from jax.experimental.pallas import tpu_sc as plsc

NUM_LATENT_CODES = 100000
LATENT_CODE_DIM = 64
LANES = 16
CHUNK = 128
PAIR = 2 * LATENT_CODE_DIM
HALF = NUM_LATENT_CODES // 2
TC_BLOCK = 400


def _repack_body(a_ref, b_ref, o_ref):
    o_ref[:, :LATENT_CODE_DIM] = a_ref[...]
    o_ref[:, LATENT_CODE_DIM:] = b_ref[...]


@functools.lru_cache(maxsize=None)
def _build_repack():
    return pl.pallas_call(
        _repack_body,
        grid=(HALF // TC_BLOCK,),
        in_specs=[
            pl.BlockSpec((TC_BLOCK, LATENT_CODE_DIM), lambda i: (i, 0)),
            pl.BlockSpec(
                (TC_BLOCK, LATENT_CODE_DIM),
                lambda i: (i + HALF // TC_BLOCK, 0),
            ),
        ],
        out_specs=pl.BlockSpec((TC_BLOCK, PAIR), lambda i: (i, 0)),
        out_shape=jax.ShapeDtypeStruct((HALF, PAIR), jnp.float32),
    )


@functools.lru_cache(maxsize=None)
def _build_sc(batch, d_in):
    d_out = d_in - 1 + LATENT_CODE_DIM
    info = plsc.get_sparse_core_info()
    num_workers = info.num_cores * info.num_subcores
    rows_per_w = batch // num_workers
    n_chunks = rows_per_w // CHUNK
    n_groups = CHUNK // LANES
    scale = float(NUM_LATENT_CODES - 1)

    mesh = plsc.VectorSubcoreMesh(core_axis_name="c", subcore_axis_name="s")

    @functools.partial(
        pl.kernel,
        mesh=mesh,
        out_type=jax.ShapeDtypeStruct((batch, d_out), jnp.float32),
        scratch_types=[
            pltpu.VMEM((CHUNK, d_out), jnp.float32),
            pltpu.VMEM((CHUNK, LATENT_CODE_DIM), jnp.float32),
            pltpu.VMEM((CHUNK, PAIR), jnp.float32),
            pltpu.VMEM((CHUNK, PAIR), jnp.float32),
            pltpu.VMEM((CHUNK,), jnp.float32),
            pltpu.VMEM((CHUNK,), jnp.int32),
            pltpu.VMEM((CHUNK,), jnp.int32),
            pltpu.VMEM((CHUNK,), jnp.int32),
            pltpu.VMEM((CHUNK,), jnp.int32),
            pltpu.VMEM((CHUNK,), jnp.float32),
            pltpu.VMEM((CHUNK,), jnp.float32),
            pltpu.SemaphoreType.DMA,
            pltpu.SemaphoreType.DMA,
        ],
    )
    def body(x_hbm, xt_hbm, table_hbm, out_hbm, out_v, x_v, f_v, c_v, t_v,
             if_v, ic_v, of_v, oc_v, w1_v, w2_v, sem1, sem2):
        wid = lax.axis_index("s") * info.num_cores + lax.axis_index("c")
        base_w = wid * rows_per_w

        for ch in range(n_chunks):
            row0 = base_w + ch * CHUNK

            pltpu.sync_copy(x_hbm.at[pl.ds(row0, CHUNK), :], x_v)
            pltpu.sync_copy(xt_hbm.at[pl.ds(row0, CHUNK)], t_v)

            def phase_idx(g, carry):
                sl = pl.ds(g * LANES, LANES)
                t = t_v[sl]
                t = jnp.minimum(jnp.maximum(t, 0.0), 1.0)
                idx = t * scale
                fl = idx.astype(jnp.int32)
                flf = fl.astype(jnp.float32)
                cl = fl + jnp.sign(idx - flf).astype(jnp.int32)
                hf = lax.div(fl, HALF)
                hc = lax.div(cl, HALF)
                if_v[sl] = fl - hf * HALF
                ic_v[sl] = cl - hc * HALF
                of_v[sl] = hf * LATENT_CODE_DIM
                oc_v[sl] = hc * LATENT_CODE_DIM
                w1_v[sl] = 1.0 - (idx - flf)
                w2_v[sl] = flf
                return carry

            lax.fori_loop(0, n_groups, phase_idx, 0)

            cp_f = pltpu.async_copy(table_hbm.at[if_v], f_v, sem1)
            cp_c = pltpu.async_copy(table_hbm.at[ic_v], c_v, sem2)
            cp_f.wait()
            cp_c.wait()

            def phase_blend(g, carry):
                sl16 = pl.ds(g * LANES, LANES)
                w1g = w1_v[sl16]
                w2g = w2_v[sl16]
                ofg = of_v[sl16]
                ocg = oc_v[sl16]
                for j in range(LANES):
                    r = g * LANES + j
                    w1 = w1g[j]
                    w2 = w2g[j]
                    pof = ofg[j]
                    poc = ocg[j]
                    for k in range(LATENT_CODE_DIM // LANES):
                        sl = pl.ds(k * LANES, LANES)
                        out_v[r, sl] = x_v[r, sl]
                    for k in range(LATENT_CODE_DIM // LANES):
                        fsl = pl.ds(pof + k * LANES, LANES)
                        csl = pl.ds(poc + k * LANES, LANES)
                        osl = pl.ds(d_in - 1 + k * LANES, LANES)
                        out_v[r, osl] = w1 * f_v[r, fsl] + w2 * c_v[r, csl]
                return carry

            lax.fori_loop(0, n_groups, phase_blend, 0)

            pltpu.sync_copy(out_v, out_hbm.at[pl.ds(row0, CHUNK), :])

    return body


def kernel(x, latent_codes):
    batch, d_in = x.shape
    table_halves = _build_repack()(latent_codes, latent_codes)
    return _build_sc(batch, d_in)(x, x[:, -1], table_halves)


# speedup vs baseline: 1.4266x; 1.4266x over previous
import functools

import jax
import jax.numpy as jnp
from jax import lax
from jax.experimental import pallas as pl
from jax.experimental.pallas import tpu as pltpu
from jax.experimental.pallas import tpu_sc as plsc

NUM_LATENT_CODES = 100000
LATENT_CODE_DIM = 64
LANES = 16
CHUNK = 128
PAIR = 2 * LATENT_CODE_DIM
HALF = NUM_LATENT_CODES // 2
TC_BLOCK = 2000


def _repack_body(a_ref, b_ref, o_ref):
    o_ref[:, :LATENT_CODE_DIM] = a_ref[...]
    o_ref[:, LATENT_CODE_DIM:] = b_ref[...]


@functools.lru_cache(maxsize=None)
def _build_repack():
    return pl.pallas_call(
        _repack_body,
        grid=(HALF // TC_BLOCK,),
        in_specs=[
            pl.BlockSpec((TC_BLOCK, LATENT_CODE_DIM), lambda i: (i, 0)),
            pl.BlockSpec(
                (TC_BLOCK, LATENT_CODE_DIM),
                lambda i: (i + HALF // TC_BLOCK, 0),
            ),
        ],
        out_specs=pl.BlockSpec((TC_BLOCK, PAIR), lambda i: (i, 0)),
        out_shape=jax.ShapeDtypeStruct((HALF, PAIR), jnp.float32),
    )


@functools.lru_cache(maxsize=None)
def _build_sc(batch, d_in):
    d_out = d_in - 1 + LATENT_CODE_DIM
    info = plsc.get_sparse_core_info()
    num_workers = info.num_cores * info.num_subcores
    rows_per_w = batch // num_workers
    n_chunks = rows_per_w // CHUNK
    n_groups = CHUNK // LANES
    scale = float(NUM_LATENT_CODES - 1)

    mesh = plsc.VectorSubcoreMesh(core_axis_name="c", subcore_axis_name="s")

    @functools.partial(
        pl.kernel,
        mesh=mesh,
        out_type=jax.ShapeDtypeStruct((batch, d_out), jnp.float32),
        scratch_types=[
            pltpu.VMEM((CHUNK, d_out), jnp.float32),
            pltpu.VMEM((2, CHUNK, LATENT_CODE_DIM), jnp.float32),
            pltpu.VMEM((2, CHUNK, PAIR), jnp.float32),
            pltpu.VMEM((2, CHUNK, PAIR), jnp.float32),
            pltpu.VMEM((CHUNK,), jnp.float32),
            pltpu.VMEM((2, CHUNK), jnp.int32),
            pltpu.VMEM((2, CHUNK), jnp.int32),
            pltpu.VMEM((2, CHUNK), jnp.int32),
            pltpu.VMEM((2, CHUNK), jnp.int32),
            pltpu.VMEM((2, CHUNK), jnp.float32),
            pltpu.VMEM((2, CHUNK), jnp.float32),
            pltpu.SemaphoreType.DMA,
            pltpu.SemaphoreType.DMA,
        ],
    )
    def body(x_hbm, xt_hbm, table_hbm, out_hbm, out_v, x_v, f_v, c_v, t_v,
             if_v, ic_v, of_v, oc_v, w1_v, w2_v, sem1, sem2):
        wid = lax.axis_index("s") * info.num_cores + lax.axis_index("c")
        base_w = wid * rows_per_w

        def phase_idx_for(b, carry_unused):
            def phase_idx(g, carry):
                sl = pl.ds(g * LANES, LANES)
                t = t_v[sl]
                t = jnp.minimum(jnp.maximum(t, 0.0), 1.0)
                idx = t * scale
                fl = idx.astype(jnp.int32)
                flf = fl.astype(jnp.float32)
                cl = fl + jnp.sign(idx - flf).astype(jnp.int32)
                hf = lax.div(fl, HALF)
                hc = lax.div(cl, HALF)
                if_v[b, sl] = fl - hf * HALF
                ic_v[b, sl] = cl - hc * HALF
                of_v[b, sl] = hf * LATENT_CODE_DIM
                oc_v[b, sl] = hc * LATENT_CODE_DIM
                w1_v[b, sl] = 1.0 - (idx - flf)
                w2_v[b, sl] = flf
                return carry
            lax.fori_loop(0, n_groups, phase_idx, 0)

        def blend_and_store(b, row0):
            def phase_blend(g, carry):
                sl16 = pl.ds(g * LANES, LANES)
                w1g = w1_v[b, sl16]
                w2g = w2_v[b, sl16]
                ofg = of_v[b, sl16]
                ocg = oc_v[b, sl16]
                for j in range(LANES):
                    r = g * LANES + j
                    w1 = w1g[j]
                    w2 = w2g[j]
                    pof = ofg[j]
                    poc = ocg[j]
                    for k in range(LATENT_CODE_DIM // LANES):
                        sl = pl.ds(k * LANES, LANES)
                        out_v[r, sl] = x_v[b, r, sl]
                    for k in range(LATENT_CODE_DIM // LANES):
                        fsl = pl.ds(pof + k * LANES, LANES)
                        csl = pl.ds(poc + k * LANES, LANES)
                        osl = pl.ds(d_in - 1 + k * LANES, LANES)
                        out_v[r, osl] = (
                            w1 * f_v[b, r, fsl] + w2 * c_v[b, r, csl])
                return carry
            lax.fori_loop(0, n_groups, phase_blend, 0)
            pltpu.sync_copy(out_v, out_hbm.at[pl.ds(row0, CHUNK), :])

        for ch in range(n_chunks):
            b = ch % 2
            row0 = base_w + ch * CHUNK
            pltpu.sync_copy(x_hbm.at[pl.ds(row0, CHUNK), :], x_v.at[b])
            pltpu.sync_copy(xt_hbm.at[pl.ds(row0, CHUNK)], t_v)
            phase_idx_for(b, None)
            cp_f = pltpu.async_copy(table_hbm.at[if_v.at[b]], f_v.at[b], sem1)
            cp_c = pltpu.async_copy(table_hbm.at[ic_v.at[b]], c_v.at[b], sem2)
            if ch > 0:
                blend_and_store(1 - b, base_w + (ch - 1) * CHUNK)
            cp_f.wait()
            cp_c.wait()

        blend_and_store((n_chunks - 1) % 2, base_w + (n_chunks - 1) * CHUNK)

    return body


def kernel(x, latent_codes):
    batch, d_in = x.shape
    table_halves = _build_repack()(latent_codes, latent_codes)
    return _build_sc(batch, d_in)(x, x[:, -1], table_halves)


# speedup vs baseline: 1.6287x; 1.1417x over previous
import functools

import jax
import jax.numpy as jnp
from jax import lax
from jax.experimental import pallas as pl
from jax.experimental.pallas import tpu as pltpu
from jax.experimental.pallas import tpu_sc as plsc

NUM_LATENT_CODES = 100000
LATENT_CODE_DIM = 64
LANES = 16
CHUNK = 128


@functools.lru_cache(maxsize=None)
def _build(batch, d_in):
    d_out = d_in - 1 + LATENT_CODE_DIM
    info = plsc.get_sparse_core_info()
    num_workers = info.num_cores * info.num_subcores
    rows_per_w = batch // num_workers
    n_chunks = rows_per_w // CHUNK
    n_groups = CHUNK // LANES
    scale = float(NUM_LATENT_CODES - 1)

    mesh = plsc.VectorSubcoreMesh(core_axis_name="c", subcore_axis_name="s")

    @functools.partial(
        pl.kernel,
        mesh=mesh,
        compiler_params=pltpu.CompilerParams(use_tc_tiling_on_sc=False),
        out_type=jax.ShapeDtypeStruct((batch, d_out), jnp.float32),
        scratch_types=[
            pltpu.VMEM((2, CHUNK, d_out), jnp.float32),
            pltpu.VMEM((2, CHUNK, LATENT_CODE_DIM), jnp.float32),
            pltpu.VMEM((2, CHUNK, LATENT_CODE_DIM), jnp.float32),
            pltpu.VMEM((CHUNK,), jnp.float32),
            pltpu.VMEM((2, CHUNK), jnp.int32),
            pltpu.VMEM((2, CHUNK), jnp.int32),
            pltpu.VMEM((2, CHUNK), jnp.float32),
            pltpu.VMEM((2, CHUNK), jnp.float32),
            pltpu.SemaphoreType.DMA,
            pltpu.SemaphoreType.DMA,
        ],
    )
    def body(x_hbm, xt_hbm, table_hbm, out_hbm, out_v, f_v, c_v, t_v, if_v,
             ic_v, w1_v, w2_v, sem1, sem2):
        wid = lax.axis_index("s") * info.num_cores + lax.axis_index("c")
        base_w = wid * rows_per_w

        def phase_idx_for(b):
            def phase_idx(g, carry):
                sl = pl.ds(g * LANES, LANES)
                t = t_v[sl]
                t = jnp.minimum(jnp.maximum(t, 0.0), 1.0)
                idx = t * scale
                fl = idx.astype(jnp.int32)
                flf = fl.astype(jnp.float32)
                cl = fl + jnp.sign(idx - flf).astype(jnp.int32)
                if_v[b, sl] = fl
                ic_v[b, sl] = cl
                w1_v[b, sl] = 1.0 - (idx - flf)
                w2_v[b, sl] = flf
                return carry
            lax.fori_loop(0, n_groups, phase_idx, 0)

        def blend_and_store(b, row0):
            def phase_blend(g, carry):
                sl16 = pl.ds(g * LANES, LANES)
                w1g = w1_v[b, sl16]
                w2g = w2_v[b, sl16]
                for j in range(LANES):
                    r = g * LANES + j
                    w1 = w1g[j]
                    w2 = w2g[j]
                    for k in range(LATENT_CODE_DIM // LANES):
                        sl = pl.ds(k * LANES, LANES)
                        osl = pl.ds(d_in - 1 + k * LANES, LANES)
                        out_v[b, r, osl] = (
                            w1 * f_v[b, r, sl] + w2 * c_v[b, r, sl])
                return carry
            lax.fori_loop(0, n_groups, phase_blend, 0)
            pltpu.sync_copy(out_v.at[b], out_hbm.at[pl.ds(row0, CHUNK), :])

        for ch in range(n_chunks):
            b = ch % 2
            row0 = base_w + ch * CHUNK
            pltpu.sync_copy(
                x_hbm.at[pl.ds(row0, CHUNK), :],
                out_v.at[b].at[:, pl.ds(0, d_in)],
            )
            pltpu.sync_copy(xt_hbm.at[pl.ds(row0, CHUNK)], t_v)
            phase_idx_for(b)
            cp_f = pltpu.async_copy(table_hbm.at[if_v.at[b]], f_v.at[b], sem1)
            cp_c = pltpu.async_copy(table_hbm.at[ic_v.at[b]], c_v.at[b], sem2)
            if ch > 0:
                blend_and_store(1 - b, base_w + (ch - 1) * CHUNK)
            cp_f.wait()
            cp_c.wait()

        blend_and_store((n_chunks - 1) % 2, base_w + (n_chunks - 1) * CHUNK)

    return body


def kernel(x, latent_codes):
    batch, d_in = x.shape
    return _build(batch, d_in)(x, x[:, -1], latent_codes)


# speedup vs baseline: 1.6919x; 1.0388x over previous
import functools

import jax
import jax.numpy as jnp
from jax import lax
from jax.experimental import pallas as pl
from jax.experimental.pallas import tpu as pltpu
from jax.experimental.pallas import tpu_sc as plsc

NUM_LATENT_CODES = 100000
LATENT_CODE_DIM = 64
LANES = 16
CHUNK = 128


@functools.lru_cache(maxsize=None)
def _build(batch, d_in):
    d_out = d_in - 1 + LATENT_CODE_DIM
    info = plsc.get_sparse_core_info()
    num_workers = info.num_cores * info.num_subcores
    rows_per_w = batch // num_workers
    n_chunks = rows_per_w // CHUNK
    n_groups = CHUNK // LANES
    scale = float(NUM_LATENT_CODES - 1)

    mesh = plsc.VectorSubcoreMesh(core_axis_name="c", subcore_axis_name="s")

    @functools.partial(
        pl.kernel,
        mesh=mesh,
        compiler_params=pltpu.CompilerParams(use_tc_tiling_on_sc=False),
        out_type=jax.ShapeDtypeStruct((batch, d_out), jnp.float32),
        scratch_types=[
            pltpu.VMEM((2, CHUNK, d_out), jnp.float32),
            pltpu.VMEM((2, CHUNK, LATENT_CODE_DIM), jnp.float32),
            pltpu.VMEM((2, CHUNK, LATENT_CODE_DIM), jnp.float32),
            pltpu.VMEM((2, CHUNK), jnp.float32),
            pltpu.VMEM((2, CHUNK), jnp.int32),
            pltpu.VMEM((2, CHUNK), jnp.int32),
            pltpu.VMEM((2, CHUNK), jnp.float32),
            pltpu.VMEM((2, CHUNK), jnp.float32),
            pltpu.SemaphoreType.DMA,
            pltpu.SemaphoreType.DMA,
            pltpu.SemaphoreType.DMA,
            pltpu.SemaphoreType.DMA,
            pltpu.SemaphoreType.DMA,
            pltpu.SemaphoreType.DMA,
        ],
    )
    def body(x_hbm, xt_hbm, table_hbm, out_hbm, out_v, f_v, c_v, t_v, if_v,
             ic_v, w1_v, w2_v, sem1, sem2, semx0, semx1, semt0, semt1):
        wid = lax.axis_index("s") * info.num_cores + lax.axis_index("c")
        base_w = wid * rows_per_w

        def phase_idx_for(b):
            def phase_idx(g, carry):
                sl = pl.ds(g * LANES, LANES)
                t = t_v[b, sl]
                t = jnp.minimum(jnp.maximum(t, 0.0), 1.0)
                idx = t * scale
                fl = idx.astype(jnp.int32)
                flf = fl.astype(jnp.float32)
                cl = fl + jnp.sign(idx - flf).astype(jnp.int32)
                if_v[b, sl] = fl
                ic_v[b, sl] = cl
                w1_v[b, sl] = 1.0 - (idx - flf)
                w2_v[b, sl] = flf
                return carry
            lax.fori_loop(0, n_groups, phase_idx, 0)

        def blend_and_store(b, row0, cp_x):
            cp_x.wait()

            def phase_blend(g, carry):
                sl16 = pl.ds(g * LANES, LANES)
                w1g = w1_v[b, sl16]
                w2g = w2_v[b, sl16]
                for j in range(LANES):
                    r = g * LANES + j
                    w1 = w1g[j]
                    w2 = w2g[j]
                    for k in range(LATENT_CODE_DIM // LANES):
                        sl = pl.ds(k * LANES, LANES)
                        osl = pl.ds(d_in - 1 + k * LANES, LANES)
                        out_v[b, r, osl] = (
                            w1 * f_v[b, r, sl] + w2 * c_v[b, r, sl])
                return carry
            lax.fori_loop(0, n_groups, phase_blend, 0)
            pltpu.sync_copy(out_v.at[b], out_hbm.at[pl.ds(row0, CHUNK), :])

        semx = (semx0, semx1)
        semt = (semt0, semt1)
        cp_t = {0: pltpu.async_copy(
            xt_hbm.at[pl.ds(base_w, CHUNK)], t_v.at[0], semt[0])}
        cp_x = {}
        for ch in range(n_chunks):
            b = ch % 2
            row0 = base_w + ch * CHUNK
            cp_x[ch] = pltpu.async_copy(
                x_hbm.at[pl.ds(row0, CHUNK), :],
                out_v.at[b].at[:, pl.ds(0, d_in)],
                semx[b],
            )
            if ch + 1 < n_chunks:
                cp_t[ch + 1] = pltpu.async_copy(
                    xt_hbm.at[pl.ds(row0 + CHUNK, CHUNK)],
                    t_v.at[1 - b],
                    semt[1 - b],
                )
            cp_t[ch].wait()
            phase_idx_for(b)
            cp_f = pltpu.async_copy(table_hbm.at[if_v.at[b]], f_v.at[b], sem1)
            cp_c = pltpu.async_copy(table_hbm.at[ic_v.at[b]], c_v.at[b], sem2)
            if ch > 0:
                blend_and_store(
                    1 - b, base_w + (ch - 1) * CHUNK, cp_x[ch - 1])
            cp_f.wait()
            cp_c.wait()

        blend_and_store(
            (n_chunks - 1) % 2,
            base_w + (n_chunks - 1) * CHUNK,
            cp_x[n_chunks - 1],
        )

    return body


def kernel(x, latent_codes):
    batch, d_in = x.shape
    return _build(batch, d_in)(x, x[:, -1], latent_codes)
